# baseline (speedup 1.0000x reference)
import functools

import jax
import jax.numpy as jnp
from jax import lax
from jax.experimental import pallas as pl
from jax.experimental.pallas import tpu as pltpu
from jax.experimental.pallas import tpu_sc as plsc


def _sc_window_copy(table, Q, K, offset):
    D = table.shape[1]
    info = plsc.get_sparse_core_info()
    NC = info.num_cores
    NW = info.num_cores * info.num_subcores
    mesh = plsc.VectorSubcoreMesh(core_axis_name="c", subcore_axis_name="s")

    row_groups = 4
    rpt = Q // row_groups
    col_groups = NW // row_groups
    cpt = K // col_groups
    span = cpt + rpt - 1
    span_al = ((span + 7) // 8) * 8

    table_flat = table.reshape(-1)

    @functools.partial(
        pl.kernel,
        out_type=jax.ShapeDtypeStruct((Q * K * D,), table.dtype),
        mesh=mesh,
        scratch_types=[
            pltpu.VMEM((span_al * D,), table.dtype),
            pltpu.SemaphoreType.DMA,
            pltpu.SemaphoreType.DMA,
        ],
    )
    def copy_kernel(table_hbm, out_hbm, buf, in_sem, out_sem):
        wid = lax.axis_index("s") * NC + lax.axis_index("c")
        g = wid % col_groups
        r = wid // col_groups
        block = g * cpt
        row0 = r * rpt
        lo = block + offset - (row0 + rpt - 1)
        lo_al = (lo // 8) * 8

        pltpu.async_copy(
            table_hbm.at[pl.ds(lo_al * D, span_al * D)], buf, in_sem
        ).wait()

        outs = []
        for ii in range(rpt):
            i = row0 + ii
            src_off = (block + offset - i - lo_al) * D
            dst_off = (i * K + block) * D
            outs.append(
                pltpu.async_copy(
                    buf.at[pl.ds(src_off, cpt * D)],
                    out_hbm.at[pl.ds(dst_off, cpt * D)],
                    out_sem,
                )
            )
        for h in outs:
            h.wait()

    return copy_kernel(table_flat).reshape(Q, K, D)


def kernel(q, k, embed_weight):
    Q = q.shape[0]
    K = k.shape[0]
    max_len = embed_weight.shape[0]
    offset = max_len // 2 + max_len % 2
    return _sc_window_copy(embed_weight, Q, K, offset)


# speedup vs baseline: 1.0423x; 1.0423x over previous
import functools

import jax
import jax.numpy as jnp
from jax import lax
from jax.experimental import pallas as pl
from jax.experimental.pallas import tpu as pltpu
from jax.experimental.pallas import tpu_sc as plsc


def _sc_window_copy(table, Q, K, offset):
    D = table.shape[1]
    info = plsc.get_sparse_core_info()
    NC = info.num_cores
    NW = info.num_cores * info.num_subcores
    mesh = plsc.VectorSubcoreMesh(core_axis_name="c", subcore_axis_name="s")

    row_groups = 2
    rpt = Q // row_groups
    col_groups = NW // row_groups
    cpt = K // col_groups
    span = cpt + rpt - 1
    span_al = ((span + 7) // 8) * 8

    table_flat = table.reshape(-1)

    @functools.partial(
        pl.kernel,
        out_type=jax.ShapeDtypeStruct((Q * K * D,), table.dtype),
        mesh=mesh,
        scratch_types=[
            pltpu.VMEM((span_al * D,), table.dtype),
            pltpu.SemaphoreType.DMA,
            pltpu.SemaphoreType.DMA,
        ],
    )
    def copy_kernel(table_hbm, out_hbm, buf, in_sem, out_sem):
        wid = lax.axis_index("s") * NC + lax.axis_index("c")
        g = wid % col_groups
        r = wid // col_groups
        block = g * cpt
        row0 = r * rpt
        lo = block + offset - (row0 + rpt - 1)
        lo_al = (lo // 8) * 8

        pltpu.async_copy(
            table_hbm.at[pl.ds(lo_al * D, span_al * D)], buf, in_sem
        ).wait()

        outs = []
        for ii in range(rpt):
            i = row0 + ii
            src_off = (block + offset - i - lo_al) * D
            dst_off = (i * K + block) * D
            outs.append(
                pltpu.async_copy(
                    buf.at[pl.ds(src_off, cpt * D)],
                    out_hbm.at[pl.ds(dst_off, cpt * D)],
                    out_sem,
                )
            )
        for h in outs:
            h.wait()

    return copy_kernel(table_flat).reshape(Q, K, D)


def kernel(q, k, embed_weight):
    Q = q.shape[0]
    K = k.shape[0]
    max_len = embed_weight.shape[0]
    offset = max_len // 2 + max_len % 2
    return _sc_window_copy(embed_weight, Q, K, offset)


# speedup vs baseline: 1.0486x; 1.0060x over previous
import functools

import jax
import jax.numpy as jnp
from jax import lax
from jax.experimental import pallas as pl
from jax.experimental.pallas import tpu as pltpu
from jax.experimental.pallas import tpu_sc as plsc


def _sc_window_copy(table, Q, K, offset):
    D = table.shape[1]
    info = plsc.get_sparse_core_info()
    NC = info.num_cores
    NW = info.num_cores * info.num_subcores
    mesh = plsc.VectorSubcoreMesh(core_axis_name="c", subcore_axis_name="s")

    row_groups = 1
    rpt = Q // row_groups
    col_groups = NW // row_groups
    cpt = K // col_groups
    span = cpt + rpt - 1
    span_al = ((span + 7) // 8) * 8

    table_flat = table.reshape(-1)

    @functools.partial(
        pl.kernel,
        out_type=jax.ShapeDtypeStruct((Q * K * D,), table.dtype),
        mesh=mesh,
        scratch_types=[
            pltpu.VMEM((span_al * D,), table.dtype),
            pltpu.SemaphoreType.DMA,
            pltpu.SemaphoreType.DMA,
        ],
    )
    def copy_kernel(table_hbm, out_hbm, buf, in_sem, out_sem):
        wid = lax.axis_index("s") * NC + lax.axis_index("c")
        g = wid % col_groups
        r = wid // col_groups
        block = g * cpt
        row0 = r * rpt
        lo = block + offset - (row0 + rpt - 1)
        lo_al = (lo // 8) * 8

        pltpu.async_copy(
            table_hbm.at[pl.ds(lo_al * D, span_al * D)], buf, in_sem
        ).wait()

        outs = []
        for ii in range(rpt):
            i = row0 + ii
            src_off = (block + offset - i - lo_al) * D
            dst_off = (i * K + block) * D
            outs.append(
                pltpu.async_copy(
                    buf.at[pl.ds(src_off, cpt * D)],
                    out_hbm.at[pl.ds(dst_off, cpt * D)],
                    out_sem,
                )
            )
        for h in outs:
            h.wait()

    return copy_kernel(table_flat).reshape(Q, K, D)


def kernel(q, k, embed_weight):
    Q = q.shape[0]
    K = k.shape[0]
    max_len = embed_weight.shape[0]
    offset = max_len // 2 + max_len % 2
    return _sc_window_copy(embed_weight, Q, K, offset)


# speedup vs baseline: 1.0593x; 1.0102x over previous
import functools

import jax
import jax.numpy as jnp
from jax import lax
from jax.experimental import pallas as pl
from jax.experimental.pallas import tpu as pltpu
from jax.experimental.pallas import tpu_sc as plsc


def _sc_window_copy(table, Q, K, offset):
    D = table.shape[1]
    info = plsc.get_sparse_core_info()
    NC = info.num_cores
    NW = info.num_cores * info.num_subcores
    mesh = plsc.VectorSubcoreMesh(core_axis_name="c", subcore_axis_name="s")

    row_groups = 1
    rpt = Q // row_groups
    col_groups = NW // row_groups
    cpt = K // col_groups
    span = cpt + rpt - 1
    span_al = ((span + 7) // 8) * 8

    table_flat = table.reshape(-1)

    @functools.partial(
        pl.kernel,
        out_type=jax.ShapeDtypeStruct((Q * K * D,), table.dtype),
        mesh=mesh,
        scratch_types=[
            pltpu.VMEM((span_al * D,), table.dtype),
            pltpu.SemaphoreType.DMA,
            pltpu.SemaphoreType.DMA,
        ],
    )
    def copy_kernel(table_hbm, out_hbm, buf, in_sem, out_sem):
        wid = lax.axis_index("s") * NC + lax.axis_index("c")
        g = wid % col_groups
        r = wid // col_groups
        block = g * cpt
        row0 = r * rpt
        lo = block + offset - (row0 + rpt - 1)
        lo_al = (lo // 8) * 8

        pltpu.async_copy(
            table_hbm.at[pl.ds(lo_al * D, span_al * D)], buf, in_sem
        ).wait()

        for ii in range(rpt):
            i = row0 + ii
            src_off = (block + offset - i - lo_al) * D
            dst_off = (i * K + block) * D
            pltpu.async_copy(
                buf.at[pl.ds(src_off, cpt * D)],
                out_hbm.at[pl.ds(dst_off, cpt * D)],
                out_sem,
            )
        pltpu.make_async_copy(
            table_hbm.at[pl.ds(0, rpt * cpt * D)],
            out_hbm.at[pl.ds(wid * rpt * cpt * D, rpt * cpt * D)],
            out_sem,
        ).wait()

    return copy_kernel(table_flat).reshape(Q, K, D)


def kernel(q, k, embed_weight):
    Q = q.shape[0]
    K = k.shape[0]
    max_len = embed_weight.shape[0]
    offset = max_len // 2 + max_len % 2
    return _sc_window_copy(embed_weight, Q, K, offset)
